# Initial kernel scaffold; baseline (speedup 1.0000x reference)
import functools

import jax
import jax.numpy as jnp
from jax import lax
from jax.experimental import pallas as pl
from jax.experimental.pallas import tpu as pltpu
from jax.experimental.pallas import tpu_sc as plsc

N = 10000
E = 320000
D = 128
H = 128

NC = 2
NS = 16
NW = NC * NS
EPW = E // NW
CHUNK = 80
NCHUNK = EPW // CHUNK
RPT = N // NS
ZR = 125
CNTW = 16

_sc_mesh = plsc.VectorSubcoreMesh(core_axis_name="c", subcore_axis_name="s")


def _fill2d(ref, rows, cols, val):
    per_row = cols // 16

    def body(t, carry):
        i = t // per_row
        j = (t % per_row) * 16
        ref[i, pl.ds(j, 16)] = jnp.full((16,), val, ref.dtype)
        return carry

    lax.fori_loop(0, rows * per_row, body, 0)


@functools.partial(
    pl.kernel,
    out_type=(
        jax.ShapeDtypeStruct((NC, N, D), jnp.float32),
        jax.ShapeDtypeStruct((NC, N, CNTW), jnp.float32),
    ),
    mesh=_sc_mesh,
    scratch_types=(
        pltpu.VMEM((NCHUNK, CHUNK), jnp.int32),
        pltpu.VMEM((NCHUNK, CHUNK), jnp.int32),
        pltpu.VMEM((CHUNK, D), jnp.float32),
        pltpu.VMEM((CHUNK, CNTW), jnp.float32),
        pltpu.VMEM((ZR, D), jnp.float32),
        pltpu.VMEM((RPT, CNTW), jnp.float32),
        pltpu.VMEM_SHARED((N, D), jnp.float32),
        pltpu.VMEM_SHARED((N, CNTW), jnp.float32),
        pltpu.SemaphoreType.DMA,
    ),
)
def _sc_scatter(nf, srci, dsti, sums_out, cnts_out,
                src_v, dst_v, rows_v, ones_v, zs_v, zc_v,
                sums_sh, cnts_sh, sem):
    c = lax.axis_index("c")
    s = lax.axis_index("s")
    wid = c * NS + s

    _fill2d(zs_v, ZR, D, 0.0)
    _fill2d(zc_v, RPT, CNTW, 0.0)
    _fill2d(ones_v, CHUNK, CNTW, 1.0)

    row0 = s * RPT
    for k in range(RPT // ZR):
        pltpu.sync_copy(zs_v, sums_sh.at[pl.ds(row0 + k * ZR, ZR)])
    pltpu.sync_copy(zc_v, cnts_sh.at[pl.ds(row0, RPT)])
    plsc.subcore_barrier()

    pltpu.sync_copy(srci.at[wid], src_v)
    pltpu.sync_copy(dsti.at[wid], dst_v)

    def body(i, carry):
        pltpu.async_copy(nf.at[src_v.at[i]], rows_v, sem).wait()
        pltpu.sync_copy(rows_v, sums_sh.at[dst_v.at[i]], add=True)
        pltpu.sync_copy(ones_v, cnts_sh.at[dst_v.at[i]], add=True)
        return carry

    lax.fori_loop(0, NCHUNK, body, 0)
    plsc.subcore_barrier()

    pltpu.sync_copy(sums_sh.at[pl.ds(row0, RPT)],
                    sums_out.at[c, pl.ds(row0, RPT)])
    pltpu.sync_copy(cnts_sh.at[pl.ds(row0, RPT)],
                    cnts_out.at[c, pl.ds(row0, RPT)])


@functools.partial(
    pl.kernel,
    out_type=(
        jax.ShapeDtypeStruct((E, D), jnp.float32),
        jax.ShapeDtypeStruct((E, D), jnp.float32),
    ),
    mesh=_sc_mesh,
    scratch_types=(
        pltpu.VMEM((NCHUNK, CHUNK), jnp.int32),
        pltpu.VMEM((NCHUNK, CHUNK), jnp.int32),
        pltpu.VMEM((CHUNK, D), jnp.float32),
        pltpu.VMEM((CHUNK, D), jnp.float32),
        pltpu.SemaphoreType.DMA,
        pltpu.SemaphoreType.DMA,
    ),
)
def _sc_gather(table, srci, dsti, x_out, y_out,
               src_v, dst_v, x_v, y_v, sem1, sem2):
    c = lax.axis_index("c")
    s = lax.axis_index("s")
    wid = c * NS + s
    base = wid * EPW

    pltpu.sync_copy(srci.at[wid], src_v)
    pltpu.sync_copy(dsti.at[wid], dst_v)

    def body(i, carry):
        cx = pltpu.async_copy(table.at[src_v.at[i]], x_v, sem1)
        cy = pltpu.async_copy(table.at[dst_v.at[i]], y_v, sem2)
        cx.wait()
        cy.wait()
        pltpu.sync_copy(x_v, x_out.at[pl.ds(base + i * CHUNK, CHUNK)])
        pltpu.sync_copy(y_v, y_out.at[pl.ds(base + i * CHUNK, CHUNK)])
        return carry

    lax.fori_loop(0, NCHUNK, body, 0)


def _mm_t(a, b):
    return lax.dot_general(a, b, (((1,), (1,)), ((), ())),
                           preferred_element_type=jnp.float32)


def _conv_body(ps, pc, nf, wl, bl, wr, out):
    psv = ps[...]
    pcv = pc[...]
    sums = psv[0] + psv[1]
    cnt = pcv[0, :, 0:1] + pcv[1, :, 0:1]
    mean = sums * (1.0 / jnp.maximum(cnt, 1.0))
    r = _mm_t(mean, wl[...]) + _mm_t(nf[...], wr[...]) + bl[...]
    out[...] = jnp.maximum(r, 0.0)


def _tc_conv(psums, pcnts, nf, wl, bl, wr):
    bn = 2000
    return pl.pallas_call(
        _conv_body,
        grid=(N // bn,),
        in_specs=[
            pl.BlockSpec((NC, bn, D), lambda i: (0, i, 0)),
            pl.BlockSpec((NC, bn, CNTW), lambda i: (0, i, 0)),
            pl.BlockSpec((bn, D), lambda i: (i, 0)),
            pl.BlockSpec((H, D), lambda i: (0, 0)),
            pl.BlockSpec((1, H), lambda i: (0, 0)),
            pl.BlockSpec((H, D), lambda i: (0, 0)),
        ],
        out_specs=pl.BlockSpec((bn, H), lambda i: (i, 0)),
        out_shape=jax.ShapeDtypeStruct((N, H), jnp.float32),
    )(psums, pcnts, nf, wl, bl, wr)


def _mlp_body(x, y, wa, wb, b1, w2, b2, out):
    xv = x[...]
    yv = y[...]
    h = _mm_t(xv * yv, wa[...]) + _mm_t(xv - yv, wb[...]) + b1[...]
    h = jnp.maximum(h, 0.0)
    t = _mm_t(h, w2[...]) + b2[...]
    out[...] = (1.0 / (1.0 + jnp.exp(-t)))[:, 0]


def _tc_mlp(x, y, wa, wb, b1, w2, b2):
    be = 2560
    return pl.pallas_call(
        _mlp_body,
        grid=(E // be,),
        in_specs=[
            pl.BlockSpec((be, D), lambda i: (i, 0)),
            pl.BlockSpec((be, D), lambda i: (i, 0)),
            pl.BlockSpec((H, H), lambda i: (0, 0)),
            pl.BlockSpec((H, H), lambda i: (0, 0)),
            pl.BlockSpec((1, H), lambda i: (0, 0)),
            pl.BlockSpec((1, H), lambda i: (0, 0)),
            pl.BlockSpec((1, 1), lambda i: (0, 0)),
        ],
        out_specs=pl.BlockSpec((be,), lambda i: (i,)),
        out_shape=jax.ShapeDtypeStruct((E,), jnp.float32),
    )(x, y, wa, wb, b1, w2, b2)


def kernel(node_features, edge_index, W_l, b_l, W_r, W_fc1, b_fc1, W_fc2, b_fc2):
    src = jnp.asarray(edge_index[0], jnp.int32).reshape(NW, NCHUNK, CHUNK)
    dst = jnp.asarray(edge_index[1], jnp.int32).reshape(NW, NCHUNK, CHUNK)

    psums, pcnts = _sc_scatter(node_features, src, dst)
    out = _tc_conv(psums, pcnts, node_features,
                   W_l, b_l.reshape(1, H), W_r)
    x, y = _sc_gather(out, src, dst)

    wa = W_fc1[:, :H]
    wb = W_fc1[:, H:]
    p = _tc_mlp(x, y, wa, wb, b_fc1.reshape(1, H),
                W_fc2, b_fc2.reshape(1, 1))
    return p.reshape(E, 1)


# speedup vs baseline: 2.1281x; 2.1281x over previous
import functools

import jax
import jax.numpy as jnp
from jax import lax
from jax.experimental import pallas as pl
from jax.experimental.pallas import tpu as pltpu
from jax.experimental.pallas import tpu_sc as plsc

N = 10000
E = 320000
D = 128
H = 128

NC = 2
NS = 16
NW = NC * NS
CHUNK = 128
NCHUNK = 80
EPW = NCHUNK * CHUNK
EP = NW * EPW
NP2 = 10016
STRIPE = 624
TAIL0 = NS * STRIPE
ZR = 48
HR = 80
NPAD = HR * 128

_sc_mesh = plsc.VectorSubcoreMesh(core_axis_name="c", subcore_axis_name="s")
_sc_params = pltpu.CompilerParams(use_tc_tiling_on_sc=False,
                                  needs_layout_passes=False)


def _fill2d(ref, rows, cols, val):
    per_row = cols // 16

    def body(t, carry):
        i = t // per_row
        j = (t % per_row) * 16
        ref[i, pl.ds(j, 16)] = jnp.full((16,), val, ref.dtype)
        return carry

    lax.fori_loop(0, rows * per_row, body, 0)


def _fill1d(ref, n, val):
    def body(t, carry):
        ref[pl.ds(t * 16, 16)] = jnp.full((16,), val, ref.dtype)
        return carry

    lax.fori_loop(0, n // 16, body, 0)


@functools.partial(
    pl.kernel,
    out_type=(
        jax.ShapeDtypeStruct((NC, N, D), jnp.float32),
        jax.ShapeDtypeStruct((NW, HR, D), jnp.float32),
    ),
    mesh=_sc_mesh,
    compiler_params=_sc_params,
    scratch_types=(
        pltpu.VMEM((CHUNK,), jnp.int32),
        pltpu.VMEM((CHUNK,), jnp.int32),
        pltpu.VMEM((CHUNK, D), jnp.float32),
        pltpu.VMEM((NPAD,), jnp.float32),
        pltpu.VMEM((ZR, D), jnp.float32),
        pltpu.VMEM_SHARED((NP2, D), jnp.float32),
        pltpu.SemaphoreType.DMA,
    ),
)
def _sc_scatter(nf, srci, dsti, sums_out, cnts_out,
                src_c, dst_c, rows_v, hist_v, zs_v, sums_sh, sem):
    c = lax.axis_index("c")
    s = lax.axis_index("s")
    wid = c * NS + s

    _fill2d(zs_v, ZR, D, 0.0)
    _fill1d(hist_v, NPAD, 0.0)

    row0 = s * STRIPE

    def zero_stripe(k, carry):
        pltpu.sync_copy(zs_v, sums_sh.at[pl.ds(row0 + k * ZR, ZR)])
        return carry

    lax.fori_loop(0, STRIPE // ZR, zero_stripe, 0)

    @pl.when(s == 0)
    def _zero_tail():
        pltpu.sync_copy(zs_v.at[pl.ds(0, 16)], sums_sh.at[pl.ds(TAIL0, 16)])

    plsc.subcore_barrier()

    ones16 = jnp.ones((16,), jnp.float32)

    def body(i, carry):
        pltpu.sync_copy(srci.at[wid, i], src_c)
        pltpu.sync_copy(dsti.at[wid, i], dst_c)
        pltpu.async_copy(nf.at[src_c], rows_v, sem).wait()
        pltpu.sync_copy(rows_v, sums_sh.at[dst_c], add=True)
        for j in range(CHUNK // 16):
            idx = dst_c[pl.ds(j * 16, 16)]
            plsc.addupdate_scatter(hist_v, [idx], ones16)
        return carry

    lax.fori_loop(0, NCHUNK, body, 0)

    def repack(t, carry):
        rows_v[t // 8, pl.ds((t % 8) * 16, 16)] = hist_v[pl.ds(t * 16, 16)]
        return carry

    lax.fori_loop(0, NPAD // 16, repack, 0)
    plsc.subcore_barrier()

    pltpu.sync_copy(sums_sh.at[pl.ds(row0, STRIPE)],
                    sums_out.at[c, pl.ds(row0, STRIPE)])

    @pl.when(s == 0)
    def _dump_tail():
        pltpu.sync_copy(sums_sh.at[pl.ds(TAIL0, 16)],
                        sums_out.at[c, pl.ds(TAIL0, 16)])

    pltpu.sync_copy(rows_v.at[pl.ds(0, HR)], cnts_out.at[wid])


@functools.partial(
    pl.kernel,
    out_type=(
        jax.ShapeDtypeStruct((EP, D), jnp.float32),
        jax.ShapeDtypeStruct((EP, D), jnp.float32),
    ),
    mesh=_sc_mesh,
    compiler_params=_sc_params,
    scratch_types=(
        pltpu.VMEM((CHUNK,), jnp.int32),
        pltpu.VMEM((CHUNK,), jnp.int32),
        pltpu.VMEM((CHUNK, D), jnp.float32),
        pltpu.VMEM((CHUNK, D), jnp.float32),
        pltpu.SemaphoreType.DMA,
        pltpu.SemaphoreType.DMA,
    ),
)
def _sc_gather(table, srci, dsti, x_out, y_out,
               src_c, dst_c, x_v, y_v, sem1, sem2):
    c = lax.axis_index("c")
    s = lax.axis_index("s")
    wid = c * NS + s
    base = wid * EPW

    def body(i, carry):
        pltpu.sync_copy(srci.at[wid, i], src_c)
        pltpu.sync_copy(dsti.at[wid, i], dst_c)
        cx = pltpu.async_copy(table.at[src_c], x_v, sem1)
        cy = pltpu.async_copy(table.at[dst_c], y_v, sem2)
        cx.wait()
        cy.wait()
        pltpu.sync_copy(x_v, x_out.at[pl.ds(base + i * CHUNK, CHUNK)])
        pltpu.sync_copy(y_v, y_out.at[pl.ds(base + i * CHUNK, CHUNK)])
        return carry

    lax.fori_loop(0, NCHUNK, body, 0)


def _mm_t(a, b):
    return lax.dot_general(a, b, (((1,), (1,)), ((), ())),
                           preferred_element_type=jnp.float32)


_BN = 2048


def _conv_body(ps, pc, nf, wl, bl, wr, out):
    psv = ps[...]
    sums = psv[0] + psv[1]
    pcv = pc[...]
    cnt2 = pcv[0]
    for t in range(1, NW):
        cnt2 = cnt2 + pcv[t]
    inv2 = 1.0 / jnp.maximum(cnt2, 1.0)
    invb = jnp.broadcast_to(inv2[:, None, :], (_BN // 128, 128, 128))
    invb = invb.reshape(_BN, 128)
    rowm = lax.broadcasted_iota(jnp.int32, (_BN, 128), 0) & 127
    lane = lax.broadcasted_iota(jnp.int32, (_BN, 128), 1)
    invcol = jnp.sum(jnp.where(lane == rowm, invb, 0.0), axis=1,
                     keepdims=True)
    mean = sums * invcol
    r = _mm_t(mean, wl[...]) + _mm_t(nf[...], wr[...]) + bl[...]
    out[...] = jnp.maximum(r, 0.0)


def _tc_conv(psums, pcnts, nf, wl, bl, wr):
    bn = _BN
    return pl.pallas_call(
        _conv_body,
        grid=(pl.cdiv(NP2, bn),),
        in_specs=[
            pl.BlockSpec((NC, bn, D), lambda i: (0, i, 0)),
            pl.BlockSpec((NW, bn // 128, D), lambda i: (0, i, 0)),
            pl.BlockSpec((bn, D), lambda i: (i, 0)),
            pl.BlockSpec((H, D), lambda i: (0, 0)),
            pl.BlockSpec((1, H), lambda i: (0, 0)),
            pl.BlockSpec((H, D), lambda i: (0, 0)),
        ],
        out_specs=pl.BlockSpec((bn, H), lambda i: (i, 0)),
        out_shape=jax.ShapeDtypeStruct((NP2, H), jnp.float32),
    )(psums, pcnts, nf, wl, bl, wr)


def _mlp_body(x, y, wa, wb, b1, w2, b2, out):
    xv = x[...]
    yv = y[...]
    h = _mm_t(xv * yv, wa[...]) + _mm_t(xv - yv, wb[...]) + b1[...]
    h = jnp.maximum(h, 0.0)
    t = jnp.sum(h * w2[...], axis=1) + b2[0, 0]
    out[...] = 1.0 / (1.0 + jnp.exp(-t))


def _tc_mlp(x, y, wa, wb, b1, w2, b2):
    be = 2048
    return pl.pallas_call(
        _mlp_body,
        grid=(EP // be,),
        in_specs=[
            pl.BlockSpec((be, D), lambda i: (i, 0)),
            pl.BlockSpec((be, D), lambda i: (i, 0)),
            pl.BlockSpec((H, H), lambda i: (0, 0)),
            pl.BlockSpec((H, H), lambda i: (0, 0)),
            pl.BlockSpec((1, H), lambda i: (0, 0)),
            pl.BlockSpec((1, H), lambda i: (0, 0)),
            pl.BlockSpec(memory_space=pltpu.SMEM),
        ],
        out_specs=pl.BlockSpec((be,), lambda i: (i,)),
        out_shape=jax.ShapeDtypeStruct((EP,), jnp.float32),
    )(x, y, wa, wb, b1, w2, b2)


def kernel(node_features, edge_index, W_l, b_l, W_r, W_fc1, b_fc1, W_fc2, b_fc2):
    src = jnp.asarray(edge_index[0], jnp.int32)
    dst = jnp.asarray(edge_index[1], jnp.int32)
    pad = EP - E
    src = jnp.concatenate([src, jnp.zeros((pad,), jnp.int32)])
    dst = jnp.concatenate([dst, jnp.full((pad,), N, jnp.int32)])
    src = src.reshape(NW, NCHUNK, CHUNK)
    dst = dst.reshape(NW, NCHUNK, CHUNK)

    psums, pcnts = _sc_scatter(node_features, src, dst)
    out = _tc_conv(psums, pcnts, node_features,
                   W_l, b_l.reshape(1, H), W_r)
    x, y = _sc_gather(out, src, dst)

    wa = W_fc1[:, :H]
    wb = W_fc1[:, H:]
    p = _tc_mlp(x, y, wa, wb, b_fc1.reshape(1, H),
                W_fc2, b_fc2.reshape(1, 1))
    return p[:E].reshape(E, 1)


# speedup vs baseline: 2.5175x; 1.1830x over previous
import functools

import jax
import jax.numpy as jnp
from jax import lax
from jax.experimental import pallas as pl
from jax.experimental.pallas import tpu as pltpu
from jax.experimental.pallas import tpu_sc as plsc

N = 10000
E = 320000
D = 128
H = 128

NC = 2
NS = 16
NW = NC * NS
CHUNK = 128
NCHUNK = 80
EPW = NCHUNK * CHUNK
EP = NW * EPW
NP2 = 10016
STRIPE = 624
TAIL0 = NS * STRIPE
ZR = 48
HR = 80
NPAD = HR * 128

_sc_mesh = plsc.VectorSubcoreMesh(core_axis_name="c", subcore_axis_name="s")
_sc_params = pltpu.CompilerParams(use_tc_tiling_on_sc=False,
                                  needs_layout_passes=False)


def _fill2d(ref, rows, cols, val):
    per_row = cols // 16

    def body(t, carry):
        i = t // per_row
        j = (t % per_row) * 16
        ref[i, pl.ds(j, 16)] = jnp.full((16,), val, ref.dtype)
        return carry

    lax.fori_loop(0, rows * per_row, body, 0)


def _fill1d(ref, n, val):
    def body(t, carry):
        ref[pl.ds(t * 16, 16)] = jnp.full((16,), val, ref.dtype)
        return carry

    lax.fori_loop(0, n // 16, body, 0)


@functools.partial(
    pl.kernel,
    out_type=(
        jax.ShapeDtypeStruct((NC, N, D), jnp.float32),
        jax.ShapeDtypeStruct((NW, HR, D), jnp.float32),
    ),
    mesh=_sc_mesh,
    compiler_params=_sc_params,
    scratch_types=(
        pltpu.VMEM((CHUNK,), jnp.int32),
        pltpu.VMEM((CHUNK,), jnp.int32),
        pltpu.VMEM((CHUNK,), jnp.int32),
        pltpu.VMEM((CHUNK,), jnp.int32),
        pltpu.VMEM((CHUNK, D), jnp.float32),
        pltpu.VMEM((CHUNK, D), jnp.float32),
        pltpu.VMEM((NPAD,), jnp.float32),
        pltpu.VMEM_SHARED((NP2, D), jnp.float32),
        pltpu.SemaphoreType.DMA,
        pltpu.SemaphoreType.DMA,
    ),
)
def _sc_scatter(nf, srci, dsti, sums_out, cnts_out,
                src_a, dst_a, src_b, dst_b, rows_a, rows_b,
                hist_v, sums_sh, sem_a, sem_b):
    c = lax.axis_index("c")
    s = lax.axis_index("s")
    wid = c * NS + s

    _fill2d(rows_a, CHUNK, D, 0.0)
    _fill1d(hist_v, NPAD, 0.0)

    row0 = s * STRIPE

    def zero_stripe(k, carry):
        pltpu.sync_copy(rows_a, sums_sh.at[pl.ds(row0 + k * CHUNK, CHUNK)])
        return carry

    lax.fori_loop(0, 4, zero_stripe, 0)
    pltpu.sync_copy(rows_a.at[pl.ds(0, STRIPE - 4 * CHUNK)],
                    sums_sh.at[pl.ds(row0 + 4 * CHUNK, STRIPE - 4 * CHUNK)])

    @pl.when(s == 0)
    def _zero_tail():
        pltpu.sync_copy(rows_a.at[pl.ds(0, 16)], sums_sh.at[pl.ds(TAIL0, 16)])

    plsc.subcore_barrier()

    ones16 = jnp.ones((16,), jnp.float32)

    pltpu.sync_copy(srci.at[wid, 0], src_a)
    pltpu.sync_copy(dsti.at[wid, 0], dst_a)
    pltpu.async_copy(nf.at[src_a], rows_a, sem_a)
    pltpu.sync_copy(srci.at[wid, 1], src_b)
    pltpu.sync_copy(dsti.at[wid, 1], dst_b)

    bufs = ((src_a, dst_a, rows_a, sem_a), (src_b, dst_b, rows_b, sem_b))

    def consume(j, cur, nxt):
        cur_s, cur_d, cur_rows, cur_sem = cur
        nxt_s, nxt_d, nxt_rows, nxt_sem = nxt

        @pl.when(j + 1 < NCHUNK)
        def _fire_next():
            pltpu.async_copy(nf.at[nxt_s], nxt_rows, nxt_sem)

        pltpu.make_async_copy(nf.at[cur_s], cur_rows, cur_sem).wait()
        pltpu.sync_copy(cur_rows, sums_sh.at[cur_d], add=True)
        for jj in range(CHUNK // 16):
            idx = cur_d[pl.ds(jj * 16, 16)]
            plsc.addupdate_scatter(hist_v, [idx], ones16)

        @pl.when(j + 2 < NCHUNK)
        def _prefetch_idx():
            pltpu.sync_copy(srci.at[wid, j + 2], cur_s)
            pltpu.sync_copy(dsti.at[wid, j + 2], cur_d)

    def body(k, carry):
        consume(2 * k, bufs[0], bufs[1])
        consume(2 * k + 1, bufs[1], bufs[0])
        return carry

    lax.fori_loop(0, NCHUNK // 2, body, 0)

    def repack(t, carry):
        rows_a[t // 8, pl.ds((t % 8) * 16, 16)] = hist_v[pl.ds(t * 16, 16)]
        return carry

    lax.fori_loop(0, NPAD // 16, repack, 0)
    plsc.subcore_barrier()

    pltpu.sync_copy(sums_sh.at[pl.ds(row0, STRIPE)],
                    sums_out.at[c, pl.ds(row0, STRIPE)])

    @pl.when(s == 0)
    def _dump_tail():
        pltpu.sync_copy(sums_sh.at[pl.ds(TAIL0, 16)],
                        sums_out.at[c, pl.ds(TAIL0, 16)])

    pltpu.sync_copy(rows_a.at[pl.ds(0, HR)], cnts_out.at[wid])


@functools.partial(
    pl.kernel,
    out_type=(
        jax.ShapeDtypeStruct((EP, D), jnp.float32),
        jax.ShapeDtypeStruct((EP, D), jnp.float32),
    ),
    mesh=_sc_mesh,
    compiler_params=_sc_params,
    scratch_types=(
        pltpu.VMEM((NCHUNK, CHUNK), jnp.int32),
        pltpu.VMEM((NCHUNK, CHUNK), jnp.int32),
        pltpu.VMEM((CHUNK, D), jnp.float32),
        pltpu.VMEM((CHUNK, D), jnp.float32),
        pltpu.VMEM((CHUNK, D), jnp.float32),
        pltpu.VMEM((CHUNK, D), jnp.float32),
        pltpu.SemaphoreType.DMA,
        pltpu.SemaphoreType.DMA,
        pltpu.SemaphoreType.DMA,
        pltpu.SemaphoreType.DMA,
    ),
)
def _sc_gather(table, srci, dsti, x_out, y_out,
               src_v, dst_v, x_a, y_a, x_b, y_b,
               gsem_a, gsem_b, wsem_a, wsem_b):
    c = lax.axis_index("c")
    s = lax.axis_index("s")
    wid = c * NS + s
    base = wid * EPW

    pltpu.sync_copy(srci.at[wid], src_v)
    pltpu.sync_copy(dsti.at[wid], dst_v)

    pltpu.async_copy(table.at[src_v.at[0]], x_a, gsem_a)
    pltpu.async_copy(table.at[dst_v.at[0]], y_a, gsem_a)

    bufs = ((x_a, y_a, gsem_a, wsem_a), (x_b, y_b, gsem_b, wsem_b))

    def consume(j, cur, nxt, first=False, last=False):
        cur_x, cur_y, cur_g, cur_w = cur
        nxt_x, nxt_y, nxt_g, nxt_w = nxt

        if not first:
            pltpu.make_async_copy(nxt_x, x_out.at[pl.ds(base, CHUNK)],
                                  nxt_w).wait()
            pltpu.make_async_copy(nxt_y, y_out.at[pl.ds(base, CHUNK)],
                                  nxt_w).wait()

        if not last:
            pltpu.async_copy(table.at[src_v.at[j + 1]], nxt_x, nxt_g)
            pltpu.async_copy(table.at[dst_v.at[j + 1]], nxt_y, nxt_g)

        pltpu.make_async_copy(table.at[src_v.at[0]], cur_x, cur_g).wait()
        pltpu.make_async_copy(table.at[dst_v.at[0]], cur_y, cur_g).wait()

        off = base + j * CHUNK
        pltpu.async_copy(cur_x, x_out.at[pl.ds(off, CHUNK)], cur_w)
        pltpu.async_copy(cur_y, y_out.at[pl.ds(off, CHUNK)], cur_w)

    consume(0, bufs[0], bufs[1], first=True)

    def body(k, carry):
        j = 2 * k + 1
        consume(j, bufs[1], bufs[0])
        consume(j + 1, bufs[0], bufs[1])
        return carry

    lax.fori_loop(0, (NCHUNK - 2) // 2, body, 0)

    consume(NCHUNK - 1, bufs[1], bufs[0], last=True)
    pltpu.make_async_copy(x_b, x_out.at[pl.ds(base, CHUNK)], wsem_b).wait()
    pltpu.make_async_copy(y_b, y_out.at[pl.ds(base, CHUNK)], wsem_b).wait()


def _mm_t(a, b):
    return lax.dot_general(a, b, (((1,), (1,)), ((), ())),
                           preferred_element_type=jnp.float32)


_BN = 2048


def _conv_body(ps, pc, nf, wl, bl, wr, out):
    psv = ps[...]
    sums = psv[0] + psv[1]
    pcv = pc[...]
    cnt2 = pcv[0]
    for t in range(1, NW):
        cnt2 = cnt2 + pcv[t]
    inv2 = 1.0 / jnp.maximum(cnt2, 1.0)
    invb = jnp.broadcast_to(inv2[:, None, :], (_BN // 128, 128, 128))
    invb = invb.reshape(_BN, 128)
    rowm = lax.broadcasted_iota(jnp.int32, (_BN, 128), 0) & 127
    lane = lax.broadcasted_iota(jnp.int32, (_BN, 128), 1)
    invcol = jnp.sum(jnp.where(lane == rowm, invb, 0.0), axis=1,
                     keepdims=True)
    mean = sums * invcol
    r = _mm_t(mean, wl[...]) + _mm_t(nf[...], wr[...]) + bl[...]
    out[...] = jnp.maximum(r, 0.0)


def _tc_conv(psums, pcnts, nf, wl, bl, wr):
    bn = _BN
    return pl.pallas_call(
        _conv_body,
        grid=(pl.cdiv(NP2, bn),),
        in_specs=[
            pl.BlockSpec((NC, bn, D), lambda i: (0, i, 0)),
            pl.BlockSpec((NW, bn // 128, D), lambda i: (0, i, 0)),
            pl.BlockSpec((bn, D), lambda i: (i, 0)),
            pl.BlockSpec((H, D), lambda i: (0, 0)),
            pl.BlockSpec((1, H), lambda i: (0, 0)),
            pl.BlockSpec((H, D), lambda i: (0, 0)),
        ],
        out_specs=pl.BlockSpec((bn, H), lambda i: (i, 0)),
        out_shape=jax.ShapeDtypeStruct((NP2, H), jnp.float32),
    )(psums, pcnts, nf, wl, bl, wr)


def _mlp_body(x, y, wa, wb, b1, w2, b2, out):
    xv = x[...]
    yv = y[...]
    h = _mm_t(xv * yv, wa[...]) + _mm_t(xv - yv, wb[...]) + b1[...]
    h = jnp.maximum(h, 0.0)
    t = jnp.sum(h * w2[...], axis=1) + b2[0, 0]
    out[...] = 1.0 / (1.0 + jnp.exp(-t))


def _tc_mlp(x, y, wa, wb, b1, w2, b2):
    be = 2048
    return pl.pallas_call(
        _mlp_body,
        grid=(EP // be,),
        in_specs=[
            pl.BlockSpec((be, D), lambda i: (i, 0)),
            pl.BlockSpec((be, D), lambda i: (i, 0)),
            pl.BlockSpec((H, H), lambda i: (0, 0)),
            pl.BlockSpec((H, H), lambda i: (0, 0)),
            pl.BlockSpec((1, H), lambda i: (0, 0)),
            pl.BlockSpec((1, H), lambda i: (0, 0)),
            pl.BlockSpec(memory_space=pltpu.SMEM),
        ],
        out_specs=pl.BlockSpec((be,), lambda i: (i,)),
        out_shape=jax.ShapeDtypeStruct((EP,), jnp.float32),
    )(x, y, wa, wb, b1, w2, b2)


def kernel(node_features, edge_index, W_l, b_l, W_r, W_fc1, b_fc1, W_fc2, b_fc2):
    src = jnp.asarray(edge_index[0], jnp.int32)
    dst = jnp.asarray(edge_index[1], jnp.int32)
    pad = EP - E
    src = jnp.concatenate([src, jnp.zeros((pad,), jnp.int32)])
    dst = jnp.concatenate([dst, jnp.full((pad,), N, jnp.int32)])
    src = src.reshape(NW, NCHUNK, CHUNK)
    dst = dst.reshape(NW, NCHUNK, CHUNK)

    psums, pcnts = _sc_scatter(node_features, src, dst)
    out = _tc_conv(psums, pcnts, node_features,
                   W_l, b_l.reshape(1, H), W_r)
    x, y = _sc_gather(out, src, dst)

    wa = W_fc1[:, :H]
    wb = W_fc1[:, H:]
    p = _tc_mlp(x, y, wa, wb, b_fc1.reshape(1, H),
                W_fc2, b_fc2.reshape(1, 1))
    return p[:E].reshape(E, 1)
